# Initial kernel scaffold; baseline (speedup 1.0000x reference)
import jax
import jax.numpy as jnp
from jax.experimental import pallas as pl


def kernel(node_feat, edge_index, edge_type, graph_ids, source_idx, target_idx, target_rel, rel_table, W_i_node, W_i_edge, W_h_node, IA1, IA2, W_h_edge_0, A1_0, A2_0, W_h_edge_1, A1_1, A2_1, W_o, comm_mlp, W_h_node_0, W_h_node_1, lin1_w, lin1_b, lin2_w, lin2_b):
    raise NotImplementedError("write your pallas kernel here")


# speedup vs baseline: 2.5392x; 2.5392x over previous
import functools

import jax
import jax.numpy as jnp
from jax import lax
from jax.experimental import pallas as pl
from jax.experimental.pallas import tpu as pltpu
from jax.experimental.pallas import tpu_sc as plsc

N, E, B, D, R = 10000, 160000, 50, 128, 200
NC, NS = 2, 16
NW = NC * NS
CH = 128
NCHUNK = E // CH
NP = 10240
RPT = NP // NS
NEG = -jnp.inf


def _cdiv(a, b):
    return (a + b - 1) // b


def _mk_sc_gather(V, Dt, nbuf):
    mesh = plsc.VectorSubcoreMesh(core_axis_name="c", subcore_axis_name="s", num_cores=NC, num_subcores=NS)
    K = _cdiv(NCHUNK, NW)
    KG = _cdiv(K, nbuf)

    @functools.partial(
        pl.kernel, mesh=mesh,
        out_type=jax.ShapeDtypeStruct((NCHUNK * CH, Dt), jnp.float32),
        scratch_types=(
            pltpu.VMEM((nbuf, CH), jnp.int32),
            pltpu.VMEM((nbuf, CH, Dt), jnp.float32),
            pltpu.SemaphoreType.DMA,
            pltpu.SemaphoreType.DMA,
        ),
    )
    def k(table_hbm, idx_hbm, out_hbm, idxv, rows, gsem, osem):
        wid = lax.axis_index("s") * NC + lax.axis_index("c")

        def group(kg, carry):
            base = kg * nbuf
            for b in range(nbuf):
                g = (base + b) * NW + wid

                @pl.when(g < NCHUNK)
                def _():
                    pltpu.sync_copy(idx_hbm.at[pl.ds(g * CH, CH)], idxv.at[b])
                    pltpu.async_copy(table_hbm.at[idxv.at[b]], rows.at[b], gsem)
            for b in range(nbuf):
                g = (base + b) * NW + wid

                @pl.when(g < NCHUNK)
                def _():
                    pltpu.make_async_copy(
                        table_hbm.at[idxv.at[b]], rows.at[b], gsem).wait()
                    pltpu.async_copy(
                        rows.at[b], out_hbm.at[pl.ds(g * CH, CH)], osem)
            for b in range(nbuf):
                g = (base + b) * NW + wid

                @pl.when(g < NCHUNK)
                def _():
                    pltpu.make_async_copy(
                        rows.at[b], out_hbm.at[pl.ds(g * CH, CH)], osem).wait()
            return carry

        lax.fori_loop(0, KG, group, 0)

    return k


def _mk_sc_seg_add(gather_table, V, nbuf):
    mesh = plsc.VectorSubcoreMesh(core_axis_name="c", subcore_axis_name="s", num_cores=NC, num_subcores=NS)
    K = _cdiv(NCHUNK, NW)
    KG = _cdiv(K, nbuf)
    scratch = [
        pltpu.VMEM((nbuf, CH), jnp.int32),
        pltpu.VMEM((nbuf, CH, D), jnp.float32),
        pltpu.VMEM_SHARED((NP, D), jnp.float32),
        pltpu.SemaphoreType.DMA,
    ]
    if gather_table:
        scratch.append(pltpu.VMEM((nbuf, CH), jnp.int32))

    @functools.partial(
        pl.kernel, mesh=mesh,
        out_type=jax.ShapeDtypeStruct((2 * NP, D), jnp.float32),
        scratch_types=tuple(scratch),
    )
    def k(src_hbm, sidx_hbm, *rest):
        if gather_table:
            gidx_hbm, z_hbm, out_hbm, idxv, valv, acc, vsem, gv = rest
        else:
            z_hbm, out_hbm, idxv, valv, acc, vsem = rest
        c = lax.axis_index("c")
        s = lax.axis_index("s")
        wid = s * NC + c
        r0 = s * RPT
        pltpu.sync_copy(z_hbm, acc.at[pl.ds(r0, RPT)])
        plsc.subcore_barrier()

        def group(kg, carry):
            base = kg * nbuf
            for b in range(nbuf):
                g = (base + b) * NW + wid

                @pl.when(g < NCHUNK)
                def _():
                    if gather_table:
                        pltpu.sync_copy(
                            gidx_hbm.at[pl.ds(g * CH, CH)], gv.at[b])
                        pltpu.async_copy(src_hbm.at[gv.at[b]], valv.at[b], vsem)
                    else:
                        pltpu.async_copy(
                            src_hbm.at[pl.ds(g * CH, CH)], valv.at[b], vsem)
                    pltpu.sync_copy(
                        sidx_hbm.at[pl.ds(g * CH, CH)], idxv.at[b])
            for b in range(nbuf):
                g = (base + b) * NW + wid

                @pl.when(g < NCHUNK)
                def _():
                    if gather_table:
                        pltpu.make_async_copy(
                            src_hbm.at[gv.at[b]], valv.at[b], vsem).wait()
                    else:
                        pltpu.make_async_copy(
                            src_hbm.at[pl.ds(g * CH, CH)], valv.at[b],
                            vsem).wait()
                    pltpu.sync_copy(valv.at[b], acc.at[idxv.at[b]], add=True)
            return carry

        lax.fori_loop(0, KG, group, 0)
        plsc.subcore_barrier()
        pltpu.sync_copy(acc.at[pl.ds(r0, RPT)],
                        out_hbm.at[pl.ds(c * NP + r0, RPT)])

    return k


BN = 2000
BE = 2000


def _wspec(shape):
    return pl.BlockSpec(shape, lambda i: tuple(0 for _ in shape))


def _mm(a, b):
    return jnp.dot(a.astype(jnp.bfloat16), b.astype(jnp.bfloat16),
                   preferred_element_type=jnp.float32)


def _mmx(a, b):
    return jnp.dot(a, b, preferred_element_type=jnp.float32,
                   precision=jax.lax.Precision.HIGHEST)


def _n1_body(nf_ref, win_ref, wq1_ref, wq3_ref, ia1a_ref, ia1b_ref,
             inode_ref, px_ref, rx_ref):
    nf = nf_ref[...]
    inode = jax.nn.relu(_mm(nf, win_ref[...]))
    inode_ref[...] = inode
    px_ref[...] = jnp.concatenate(
        [_mm(nf, wq1_ref[...]), _mm(inode, ia1a_ref[...])], axis=1)
    rx_ref[...] = jnp.concatenate(
        [_mm(nf, wq3_ref[...]), _mm(inode, ia1b_ref[...])], axis=1)


def _n1(nf, win, wq1, wq3, ia1a, ia1b):
    grid = (N // BN,)
    return pl.pallas_call(
        _n1_body, grid=grid,
        in_specs=[pl.BlockSpec((BN, D), lambda i: (i, 0))]
        + [_wspec((D, D))] * 5,
        out_specs=(pl.BlockSpec((BN, D), lambda i: (i, 0)),
                   pl.BlockSpec((BN, 2 * D), lambda i: (i, 0)),
                   pl.BlockSpec((BN, 2 * D), lambda i: (i, 0))),
        out_shape=(jax.ShapeDtypeStruct((N, D), jnp.float32),
                   jax.ShapeDtypeStruct((N, 2 * D), jnp.float32),
                   jax.ShapeDtypeStruct((N, 2 * D), jnp.float32)),
    )(nf, win, wq1, wq3, ia1a, ia1b)


def _e1_body(px_ref, rx_ref, rel_ref, wq2_ref, ia2_ref, out_ref):
    px = px_ref[...]
    rx = rx_ref[...]
    pre = jax.nn.relu(px[:, :D] + _mm(rel_ref[...], wq2_ref[...])
                      + rx[:, :D])
    s = _mm(jax.nn.relu(px[:, D:] + rx[:, D:]), ia2_ref[...])
    out_ref[...] = pre * jax.nn.sigmoid(s)


def _e1(px, rx, rel_e, wq2, ia2):
    grid = (E // BE,)
    return pl.pallas_call(
        _e1_body, grid=grid,
        in_specs=[pl.BlockSpec((BE, 2 * D), lambda i: (i, 0)),
                  pl.BlockSpec((BE, 2 * D), lambda i: (i, 0)),
                  pl.BlockSpec((BE, D), lambda i: (i, 0)),
                  _wspec((D, D)), _wspec((D, 1))],
        out_specs=pl.BlockSpec((BE, D), lambda i: (i, 0)),
        out_shape=jax.ShapeDtypeStruct((E, D), jnp.float32),
    )(px, rx, rel_e, wq2, ia2)


def _n2_body(inode_ref, s1_ref, s2_ref, s3_ref, w0_ref, w1_ref, w2_ref,
             w3_ref, out_ref):
    s1 = s1_ref[0] + s1_ref[1]
    s2 = s2_ref[0] + s2_ref[1]
    s3 = s3_ref[0] + s3_ref[1]
    out_ref[...] = jax.nn.relu(
        _mm(inode_ref[...], w0_ref[...]) + _mm(s1, w1_ref[...])
        + _mm(s2, w2_ref[...]) + _mm(s3, w3_ref[...]))


def _n2(inode, s1, s2, s3, w0, w1, w2, w3):
    grid = (N // BN,)
    pspec = pl.BlockSpec((2, BN, D), lambda i: (0, i, 0))
    return pl.pallas_call(
        _n2_body, grid=grid,
        in_specs=[pl.BlockSpec((BN, D), lambda i: (i, 0)),
                  pspec, pspec, pspec] + [_wspec((D, D))] * 4,
        out_specs=pl.BlockSpec((BN, D), lambda i: (i, 0)),
        out_shape=jax.ShapeDtypeStruct((N, D), jnp.float32),
    )(inode, s1, s2, s3, w0, w1, w2, w3)


def _n3_body(nh_ref, agg_ref, whn_ref, nh_out_ref):
    nh_out_ref[...] = jax.nn.relu(
        _mm(nh_ref[...] + agg_ref[0] + agg_ref[1], whn_ref[...]))


def _n3(nh, agg, whn):
    grid = (N // BN,)
    return pl.pallas_call(
        _n3_body, grid=grid,
        in_specs=[pl.BlockSpec((BN, D), lambda i: (i, 0)),
                  pl.BlockSpec((2, BN, D), lambda i: (0, i, 0)),
                  _wspec((D, D))],
        out_specs=pl.BlockSpec((BN, D), lambda i: (i, 0)),
        out_shape=jax.ShapeDtypeStruct((N, D), jnp.float32),
    )(nh, agg, whn)


def _e2_body(eh_ref, rel_ref, ng_ref, ie_ref, a1a_ref, a1b_ref, a2_ref,
             whe_ref, out_ref):
    m = _mm(eh_ref[...], a1a_ref[...]) + _mm(rel_ref[...], a1b_ref[...])
    att = jax.nn.sigmoid(_mm(jax.nn.relu(m), a2_ref[...]))
    out_ref[...] = jax.nn.relu(
        _mm(att * ng_ref[...], whe_ref[...]) + ie_ref[...])


def _e2(eh, rel_e, ng, ie, a1a, a1b, a2, whe):
    grid = (E // BE,)
    espec = pl.BlockSpec((BE, D), lambda i: (i, 0))
    return pl.pallas_call(
        _e2_body, grid=grid,
        in_specs=[espec, espec, espec, espec,
                  _wspec((D, D)), _wspec((D, D)), _wspec((D, 1)),
                  _wspec((D, D))],
        out_specs=espec,
        out_shape=jax.ShapeDtypeStruct((E, D), jnp.float32),
    )(eh, rel_e, ng, ie, a1a, a1b, a2, whe)


def _n4_body(nh_ref, agg_ref, w1_ref, w2_ref, out_ref):
    out_ref[...] = jax.nn.relu(
        _mm(nh_ref[...], w1_ref[...])
        + _mm(agg_ref[0] + agg_ref[1], w2_ref[...]))


def _n4(nh, agg, w1, w2):
    grid = (N // BN,)
    return pl.pallas_call(
        _n4_body, grid=grid,
        in_specs=[pl.BlockSpec((BN, D), lambda i: (i, 0)),
                  pl.BlockSpec((2, BN, D), lambda i: (0, i, 0)),
                  _wspec((D, D)), _wspec((D, D))],
        out_specs=pl.BlockSpec((BN, D), lambda i: (i, 0)),
        out_shape=jax.ShapeDtypeStruct((N, D), jnp.float32),
    )(nh, agg, w1, w2)


def _n5_body(ids_ref, x_ref, o_ref):
    i = pl.program_id(0)

    @pl.when(i == 0)
    def _():
        o_ref[...] = jnp.full((64, D), NEG, jnp.float32)

    x = x_ref[...]
    ids = ids_ref[...]
    for b in range(B):
        m = jnp.where(ids == b, x, NEG)
        o_ref[b, :] = jnp.maximum(o_ref[b, :], jnp.max(m, axis=0))


def _n5(ids2, node_out):
    grid = (N // BN,)
    return pl.pallas_call(
        _n5_body, grid=grid,
        in_specs=[pl.BlockSpec((BN, 1), lambda i: (i, 0)),
                  pl.BlockSpec((BN, D), lambda i: (i, 0))],
        out_specs=pl.BlockSpec((64, D), lambda i: (0, 0)),
        out_shape=jax.ShapeDtypeStruct((64, D), jnp.float32),
    )(ids2, node_out)


def _n6_body(no_ref, in_ref, g_ref, gidf_ref, selp_ref, trelp_ref, rel_ref,
             c1_ref, c2_ref, c3_ref, l1w_ref, l1b_ref, l2w_ref, l2b_ref,
             o_ref):
    selp = selp_ref[...]
    oh = (lax.broadcasted_iota(jnp.int32, (2 * 64, N), 1)
          == selp).astype(jnp.float32)
    no_sel = _mmx(oh, no_ref[...])
    in_sel = _mmx(oh, in_ref[...])
    gid_sel = _mmx(oh, gidf_ref[...])
    g = g_ref[...]
    g = jnp.where(jnp.isfinite(g), g, 0.0)
    h2 = _mm(g, c2_ref[...])
    oh_g = (lax.broadcasted_iota(jnp.int32, (2 * 64, 64), 1)
            .astype(jnp.float32) == gid_sel).astype(jnp.float32)
    h2_sel = _mmx(oh_g, h2)
    nf_sel = jax.nn.relu(_mm(no_sel, c1_ref[...]) + h2_sel
                         + _mm(in_sel, c3_ref[...]))
    oh_r = (lax.broadcasted_iota(jnp.int32, (64, R), 1)
            == trelp_ref[...]).astype(jnp.float32)
    trel = _mmx(oh_r, rel_ref[...])
    conv = jnp.tanh(nf_sel[:64] + trel - nf_sel[64:])
    o1 = _mm(conv, l1w_ref[...]) + l1b_ref[...]
    o_ref[...] = _mm(o1, l2w_ref[...]) + l2b_ref[...]


def _n6(node_out, inode, g_out, gidf, selp, trelp, rel_table,
        c1, c2, c3, l1w, l1b, l2w, l2b):
    return pl.pallas_call(
        _n6_body,
        out_shape=jax.ShapeDtypeStruct((64, 1), jnp.float32),
    )(node_out, inode, g_out, gidf, selp, trelp, rel_table,
      c1, c2, c3, l1w, l1b, l2w, l2b)


def kernel(node_feat, edge_index, edge_type, graph_ids, source_idx,
           target_idx, target_rel, rel_table, W_i_node, W_i_edge, W_h_node,
           IA1, IA2, W_h_edge_0, A1_0, A2_0, W_h_edge_1, A1_1, A2_1, W_o,
           comm_mlp, W_h_node_0, W_h_node_1, lin1_w, lin1_b, lin2_w, lin2_b):
    src2 = edge_index[0]
    dst2 = edge_index[1]
    et2 = edge_type
    zrows = jnp.zeros((RPT, D), jnp.float32)

    def halves(o):
        return jnp.stack([o[:N], o[NP:NP + N]])

    gather128 = _mk_sc_gather(N, D, 4)
    gather128r = _mk_sc_gather(R, D, 4)
    gather256 = _mk_sc_gather(N, 2 * D, 3)
    seg_add_gtab = _mk_sc_seg_add(True, N, 2)
    seg_add_dense = _mk_sc_seg_add(False, 0, 2)

    inode, pxn, rxn = _n1(node_feat, W_i_node, W_i_edge[:D],
                          W_i_edge[2 * D:], IA1[:D], IA1[D:])

    px_e = gather256(pxn, src2)
    rx_e = gather256(rxn, dst2)
    rel_e = gather128r(rel_table, et2)

    input_edge = _e1(px_e, rx_e, rel_e, W_i_edge[D:2 * D], IA2)

    s1 = halves(seg_add_gtab(node_feat, dst2, src2, zrows))
    s3 = halves(seg_add_gtab(node_feat, dst2, dst2, zrows))
    s2 = halves(seg_add_gtab(rel_table, dst2, et2, zrows))
    node_h = _n2(inode, s1, s2, s3, W_h_node[:D], W_h_node[D:2 * D],
                 W_h_node[2 * D:3 * D], W_h_node[3 * D:])

    edge_h = input_edge
    for whe, a1, a2, whn in ((W_h_edge_0, A1_0, A2_0, W_h_node_0),
                             (W_h_edge_1, A1_1, A2_1, W_h_node_1)):
        agg = halves(seg_add_dense(edge_h, dst2, zrows))
        node_h = _n3(node_h, agg, whn)
        ng_e = gather128(node_h, src2)
        edge_h = _e2(edge_h, rel_e, ng_e, input_edge, a1[:D], a1[D:], a2,
                     whe)

    aggf = halves(seg_add_dense(edge_h, dst2, zrows))
    node_out = _n4(node_h, aggf, W_o[:D], W_o[D:])

    ids2 = graph_ids.reshape(N, 1)
    g_out = _n5(ids2, node_out)
    zpad = jnp.zeros((14,), jnp.int32)
    selp = jnp.concatenate([source_idx, zpad, target_idx, zpad]) \
        .astype(jnp.int32).reshape(2 * 64, 1)
    trelp = jnp.concatenate([target_rel, zpad]).astype(jnp.int32) \
        .reshape(64, 1)
    gidf = graph_ids.astype(jnp.float32).reshape(N, 1)
    outp = _n6(node_out, inode, g_out, gidf, selp, trelp, rel_table,
               comm_mlp[:D], comm_mlp[D:2 * D], comm_mlp[2 * D:],
               lin1_w, lin1_b.reshape(1, 16), lin2_w, lin2_b.reshape(1, 1))
    return outp[:B]


# speedup vs baseline: 2.5393x; 1.0000x over previous
import functools

import jax
import jax.numpy as jnp
from jax import lax
from jax.experimental import pallas as pl
from jax.experimental.pallas import tpu as pltpu
from jax.experimental.pallas import tpu_sc as plsc

N, E, B, D, R = 10000, 160000, 50, 128, 200
NC, NS = 2, 16
NW = NC * NS
CH = 128
NCHUNK = E // CH
NP = 10240
RPT = NP // NS
NEG = -jnp.inf


def _cdiv(a, b):
    return (a + b - 1) // b


def _mk_sc_gather(V, Dt, nbuf):
    mesh = plsc.VectorSubcoreMesh(core_axis_name="c", subcore_axis_name="s", num_cores=NC, num_subcores=NS)
    K = _cdiv(NCHUNK, NW)
    KG = _cdiv(K, nbuf)

    @functools.partial(
        pl.kernel, mesh=mesh,
        out_type=jax.ShapeDtypeStruct((NCHUNK * CH, Dt), jnp.float32),
        scratch_types=(
            pltpu.VMEM((nbuf, CH), jnp.int32),
            pltpu.VMEM((nbuf, CH, Dt), jnp.float32),
            pltpu.SemaphoreType.DMA,
            pltpu.SemaphoreType.DMA,
        ),
    )
    def k(table_hbm, idx_hbm, out_hbm, idxv, rows, gsem, osem):
        wid = lax.axis_index("s") * NC + lax.axis_index("c")

        def group(kg, carry):
            base = kg * nbuf
            for b in range(nbuf):
                g = (base + b) * NW + wid

                @pl.when(g < NCHUNK)
                def _():
                    pltpu.sync_copy(idx_hbm.at[pl.ds(g * CH, CH)], idxv.at[b])
                    pltpu.async_copy(table_hbm.at[idxv.at[b]], rows.at[b], gsem)
            for b in range(nbuf):
                g = (base + b) * NW + wid

                @pl.when(g < NCHUNK)
                def _():
                    pltpu.make_async_copy(
                        table_hbm.at[idxv.at[b]], rows.at[b], gsem).wait()
                    pltpu.async_copy(
                        rows.at[b], out_hbm.at[pl.ds(g * CH, CH)], osem)
            for b in range(nbuf):
                g = (base + b) * NW + wid

                @pl.when(g < NCHUNK)
                def _():
                    pltpu.make_async_copy(
                        rows.at[b], out_hbm.at[pl.ds(g * CH, CH)], osem).wait()
            return carry

        lax.fori_loop(0, KG, group, 0)

    return k


def _mk_sc_seg_add(gather_table, V, nbuf):
    mesh = plsc.VectorSubcoreMesh(core_axis_name="c", subcore_axis_name="s", num_cores=NC, num_subcores=NS)
    K = _cdiv(NCHUNK, NW)
    KG = _cdiv(K, nbuf)
    scratch = [
        pltpu.VMEM((nbuf, CH), jnp.int32),
        pltpu.VMEM((nbuf, CH, D), jnp.float32),
        pltpu.VMEM_SHARED((NP, D), jnp.float32),
        pltpu.SemaphoreType.DMA,
    ]
    if gather_table:
        scratch.append(pltpu.VMEM((nbuf, CH), jnp.int32))

    @functools.partial(
        pl.kernel, mesh=mesh,
        out_type=jax.ShapeDtypeStruct((2 * NP, D), jnp.float32),
        scratch_types=tuple(scratch),
    )
    def k(src_hbm, sidx_hbm, *rest):
        if gather_table:
            gidx_hbm, z_hbm, out_hbm, idxv, valv, acc, vsem, gv = rest
        else:
            z_hbm, out_hbm, idxv, valv, acc, vsem = rest
        c = lax.axis_index("c")
        s = lax.axis_index("s")
        wid = s * NC + c
        r0 = s * RPT
        pltpu.sync_copy(z_hbm, acc.at[pl.ds(r0, RPT)])
        plsc.subcore_barrier()

        def group(kg, carry):
            base = kg * nbuf
            for b in range(nbuf):
                g = (base + b) * NW + wid

                @pl.when(g < NCHUNK)
                def _():
                    if gather_table:
                        pltpu.sync_copy(
                            gidx_hbm.at[pl.ds(g * CH, CH)], gv.at[b])
                        pltpu.async_copy(src_hbm.at[gv.at[b]], valv.at[b], vsem)
                    else:
                        pltpu.async_copy(
                            src_hbm.at[pl.ds(g * CH, CH)], valv.at[b], vsem)
                    pltpu.sync_copy(
                        sidx_hbm.at[pl.ds(g * CH, CH)], idxv.at[b])
            for b in range(nbuf):
                g = (base + b) * NW + wid

                @pl.when(g < NCHUNK)
                def _():
                    if gather_table:
                        pltpu.make_async_copy(
                            src_hbm.at[gv.at[b]], valv.at[b], vsem).wait()
                    else:
                        pltpu.make_async_copy(
                            src_hbm.at[pl.ds(g * CH, CH)], valv.at[b],
                            vsem).wait()
                    pltpu.sync_copy(valv.at[b], acc.at[idxv.at[b]], add=True)
            return carry

        lax.fori_loop(0, KG, group, 0)
        plsc.subcore_barrier()
        pltpu.sync_copy(acc.at[pl.ds(r0, RPT)],
                        out_hbm.at[pl.ds(c * NP + r0, RPT)])

    return k


BN = 2000
BE = 2000


def _wspec(shape):
    return pl.BlockSpec(shape, lambda i: tuple(0 for _ in shape))


def _mm(a, b):
    return jnp.dot(a.astype(jnp.bfloat16), b.astype(jnp.bfloat16),
                   preferred_element_type=jnp.float32)


def _tanh(x):
    x = jnp.clip(x, -7.90531110763549805, 7.90531110763549805)
    x2 = x * x
    p = 2.00018790482477e-13 + x2 * -2.76076847742355e-16
    p = -8.60467152213735e-11 + x2 * p
    p = 5.12229709037114e-08 + x2 * p
    p = 1.48572235717979e-05 + x2 * p
    p = 6.37261928875436e-04 + x2 * p
    p = 4.89352455891786e-03 + x2 * p
    q = 1.19825839466702e-06
    q = 1.18534705686654e-04 + x2 * q
    q = 2.26843463243900e-03 + x2 * q
    q = 4.89352518554385e-03 + x2 * q
    return jnp.where(jnp.abs(x) < 0.0004, x, (x * p) / q)


def _sig(x):
    return 1.0 / (1.0 + jnp.exp(-x))


def _mmx(a, b):
    return jnp.dot(a, b, preferred_element_type=jnp.float32,
                   precision=jax.lax.Precision.HIGHEST)


def _n1_body(nf_ref, win_ref, wq1_ref, wq3_ref, ia1a_ref, ia1b_ref,
             inode_ref, px_ref, rx_ref):
    nf = nf_ref[...]
    inode = jax.nn.relu(_mm(nf, win_ref[...]))
    inode_ref[...] = inode
    px_ref[...] = jnp.concatenate(
        [_mm(nf, wq1_ref[...]), _mm(inode, ia1a_ref[...])], axis=1)
    rx_ref[...] = jnp.concatenate(
        [_mm(nf, wq3_ref[...]), _mm(inode, ia1b_ref[...])], axis=1)


def _n1(nf, win, wq1, wq3, ia1a, ia1b):
    grid = (N // BN,)
    return pl.pallas_call(
        _n1_body, grid=grid,
        in_specs=[pl.BlockSpec((BN, D), lambda i: (i, 0))]
        + [_wspec((D, D))] * 5,
        out_specs=(pl.BlockSpec((BN, D), lambda i: (i, 0)),
                   pl.BlockSpec((BN, 2 * D), lambda i: (i, 0)),
                   pl.BlockSpec((BN, 2 * D), lambda i: (i, 0))),
        out_shape=(jax.ShapeDtypeStruct((N, D), jnp.float32),
                   jax.ShapeDtypeStruct((N, 2 * D), jnp.float32),
                   jax.ShapeDtypeStruct((N, 2 * D), jnp.float32)),
    )(nf, win, wq1, wq3, ia1a, ia1b)


def _e1_body(px_ref, rx_ref, rel_ref, wq2_ref, ia2_ref, out_ref):
    px = px_ref[...]
    rx = rx_ref[...]
    pre = jax.nn.relu(px[:, :D] + _mm(rel_ref[...], wq2_ref[...])
                      + rx[:, :D])
    s = _mm(jax.nn.relu(px[:, D:] + rx[:, D:]), ia2_ref[...])
    out_ref[...] = pre * _sig(s)


def _e1(px, rx, rel_e, wq2, ia2):
    grid = (E // BE,)
    return pl.pallas_call(
        _e1_body, grid=grid,
        in_specs=[pl.BlockSpec((BE, 2 * D), lambda i: (i, 0)),
                  pl.BlockSpec((BE, 2 * D), lambda i: (i, 0)),
                  pl.BlockSpec((BE, D), lambda i: (i, 0)),
                  _wspec((D, D)), _wspec((D, 1))],
        out_specs=pl.BlockSpec((BE, D), lambda i: (i, 0)),
        out_shape=jax.ShapeDtypeStruct((E, D), jnp.float32),
    )(px, rx, rel_e, wq2, ia2)


def _n2_body(inode_ref, s1_ref, s2_ref, s3_ref, w0_ref, w1_ref, w2_ref,
             w3_ref, out_ref):
    s1 = s1_ref[0] + s1_ref[1]
    s2 = s2_ref[0] + s2_ref[1]
    s3 = s3_ref[0] + s3_ref[1]
    out_ref[...] = jax.nn.relu(
        _mm(inode_ref[...], w0_ref[...]) + _mm(s1, w1_ref[...])
        + _mm(s2, w2_ref[...]) + _mm(s3, w3_ref[...]))


def _n2(inode, s1, s2, s3, w0, w1, w2, w3):
    grid = (N // BN,)
    pspec = pl.BlockSpec((2, BN, D), lambda i: (0, i, 0))
    return pl.pallas_call(
        _n2_body, grid=grid,
        in_specs=[pl.BlockSpec((BN, D), lambda i: (i, 0)),
                  pspec, pspec, pspec] + [_wspec((D, D))] * 4,
        out_specs=pl.BlockSpec((BN, D), lambda i: (i, 0)),
        out_shape=jax.ShapeDtypeStruct((N, D), jnp.float32),
    )(inode, s1, s2, s3, w0, w1, w2, w3)


def _n3_body(nh_ref, agg_ref, whn_ref, nh_out_ref):
    nh_out_ref[...] = jax.nn.relu(
        _mm(nh_ref[...] + agg_ref[0] + agg_ref[1], whn_ref[...]))


def _n3(nh, agg, whn):
    grid = (N // BN,)
    return pl.pallas_call(
        _n3_body, grid=grid,
        in_specs=[pl.BlockSpec((BN, D), lambda i: (i, 0)),
                  pl.BlockSpec((2, BN, D), lambda i: (0, i, 0)),
                  _wspec((D, D))],
        out_specs=pl.BlockSpec((BN, D), lambda i: (i, 0)),
        out_shape=jax.ShapeDtypeStruct((N, D), jnp.float32),
    )(nh, agg, whn)


def _e2_body(eh_ref, rel_ref, ng_ref, ie_ref, a1a_ref, a1b_ref, a2_ref,
             whe_ref, out_ref):
    m = _mm(eh_ref[...], a1a_ref[...]) + _mm(rel_ref[...], a1b_ref[...])
    att = _sig(_mm(jax.nn.relu(m), a2_ref[...]))
    out_ref[...] = jax.nn.relu(
        _mm(att * ng_ref[...], whe_ref[...]) + ie_ref[...])


def _e2(eh, rel_e, ng, ie, a1a, a1b, a2, whe):
    grid = (E // BE,)
    espec = pl.BlockSpec((BE, D), lambda i: (i, 0))
    return pl.pallas_call(
        _e2_body, grid=grid,
        in_specs=[espec, espec, espec, espec,
                  _wspec((D, D)), _wspec((D, D)), _wspec((D, 1)),
                  _wspec((D, D))],
        out_specs=espec,
        out_shape=jax.ShapeDtypeStruct((E, D), jnp.float32),
    )(eh, rel_e, ng, ie, a1a, a1b, a2, whe)


def _n4_body(nh_ref, agg_ref, w1_ref, w2_ref, out_ref):
    out_ref[...] = jax.nn.relu(
        _mm(nh_ref[...], w1_ref[...])
        + _mm(agg_ref[0] + agg_ref[1], w2_ref[...]))


def _n4(nh, agg, w1, w2):
    grid = (N // BN,)
    return pl.pallas_call(
        _n4_body, grid=grid,
        in_specs=[pl.BlockSpec((BN, D), lambda i: (i, 0)),
                  pl.BlockSpec((2, BN, D), lambda i: (0, i, 0)),
                  _wspec((D, D)), _wspec((D, D))],
        out_specs=pl.BlockSpec((BN, D), lambda i: (i, 0)),
        out_shape=jax.ShapeDtypeStruct((N, D), jnp.float32),
    )(nh, agg, w1, w2)


def _n5_body(ids_ref, x_ref, o_ref):
    i = pl.program_id(0)

    @pl.when(i == 0)
    def _():
        o_ref[...] = jnp.full((64, D), NEG, jnp.float32)

    x = x_ref[...]
    ids = ids_ref[...]
    for b in range(B):
        m = jnp.where(ids == b, x, NEG)
        o_ref[b, :] = jnp.maximum(o_ref[b, :], jnp.max(m, axis=0))


def _n5(ids2, node_out):
    grid = (N // BN,)
    return pl.pallas_call(
        _n5_body, grid=grid,
        in_specs=[pl.BlockSpec((BN, 1), lambda i: (i, 0)),
                  pl.BlockSpec((BN, D), lambda i: (i, 0))],
        out_specs=pl.BlockSpec((64, D), lambda i: (0, 0)),
        out_shape=jax.ShapeDtypeStruct((64, D), jnp.float32),
    )(ids2, node_out)


def _n6_body(no_ref, in_ref, g_ref, gidf_ref, selp_ref, trelp_ref, rel_ref,
             c1_ref, c2_ref, c3_ref, l1w_ref, l1b_ref, l2w_ref, l2b_ref,
             o_ref):
    selp = selp_ref[...]
    oh = (lax.broadcasted_iota(jnp.int32, (2 * 64, N), 1)
          == selp).astype(jnp.float32)
    no_sel = _mmx(oh, no_ref[...])
    in_sel = _mmx(oh, in_ref[...])
    gid_sel = _mmx(oh, gidf_ref[...])
    g = g_ref[...]
    g = jnp.where(jnp.isfinite(g), g, 0.0)
    h2 = _mm(g, c2_ref[...])
    oh_g = (lax.broadcasted_iota(jnp.int32, (2 * 64, 64), 1)
            .astype(jnp.float32) == gid_sel).astype(jnp.float32)
    h2_sel = _mmx(oh_g, h2)
    nf_sel = jax.nn.relu(_mm(no_sel, c1_ref[...]) + h2_sel
                         + _mm(in_sel, c3_ref[...]))
    oh_r = (lax.broadcasted_iota(jnp.int32, (64, R), 1)
            == trelp_ref[...]).astype(jnp.float32)
    trel = _mmx(oh_r, rel_ref[...])
    conv = _tanh(nf_sel[:64] + trel - nf_sel[64:])
    o1 = _mm(conv, l1w_ref[...]) + l1b_ref[...]
    o_ref[...] = _mm(o1, l2w_ref[...]) + l2b_ref[...]


def _n6(node_out, inode, g_out, gidf, selp, trelp, rel_table,
        c1, c2, c3, l1w, l1b, l2w, l2b):
    return pl.pallas_call(
        _n6_body,
        out_shape=jax.ShapeDtypeStruct((64, 1), jnp.float32),
    )(node_out, inode, g_out, gidf, selp, trelp, rel_table,
      c1, c2, c3, l1w, l1b, l2w, l2b)


def kernel(node_feat, edge_index, edge_type, graph_ids, source_idx,
           target_idx, target_rel, rel_table, W_i_node, W_i_edge, W_h_node,
           IA1, IA2, W_h_edge_0, A1_0, A2_0, W_h_edge_1, A1_1, A2_1, W_o,
           comm_mlp, W_h_node_0, W_h_node_1, lin1_w, lin1_b, lin2_w, lin2_b):
    src2 = edge_index[0]
    dst2 = edge_index[1]
    et2 = edge_type
    zrows = jnp.zeros((RPT, D), jnp.float32)

    def halves(o):
        return jnp.stack([o[:N], o[NP:NP + N]])

    gather128 = _mk_sc_gather(N, D, 4)
    gather128r = _mk_sc_gather(R, D, 4)
    gather256 = _mk_sc_gather(N, 2 * D, 3)
    seg_add_gtab = _mk_sc_seg_add(True, N, 2)
    seg_add_dense = _mk_sc_seg_add(False, 0, 2)

    inode, pxn, rxn = _n1(node_feat, W_i_node, W_i_edge[:D],
                          W_i_edge[2 * D:], IA1[:D], IA1[D:])

    px_e = gather256(pxn, src2)
    rx_e = gather256(rxn, dst2)
    rel_e = gather128r(rel_table, et2)

    input_edge = _e1(px_e, rx_e, rel_e, W_i_edge[D:2 * D], IA2)

    s1 = halves(seg_add_gtab(node_feat, dst2, src2, zrows))
    s3 = halves(seg_add_gtab(node_feat, dst2, dst2, zrows))
    s2 = halves(seg_add_gtab(rel_table, dst2, et2, zrows))
    node_h = _n2(inode, s1, s2, s3, W_h_node[:D], W_h_node[D:2 * D],
                 W_h_node[2 * D:3 * D], W_h_node[3 * D:])

    edge_h = input_edge
    for whe, a1, a2, whn in ((W_h_edge_0, A1_0, A2_0, W_h_node_0),
                             (W_h_edge_1, A1_1, A2_1, W_h_node_1)):
        agg = halves(seg_add_dense(edge_h, dst2, zrows))
        node_h = _n3(node_h, agg, whn)
        ng_e = gather128(node_h, src2)
        edge_h = _e2(edge_h, rel_e, ng_e, input_edge, a1[:D], a1[D:], a2,
                     whe)

    aggf = halves(seg_add_dense(edge_h, dst2, zrows))
    node_out = _n4(node_h, aggf, W_o[:D], W_o[D:])

    ids2 = graph_ids.reshape(N, 1)
    g_out = _n5(ids2, node_out)
    zpad = jnp.zeros((14,), jnp.int32)
    selp = jnp.concatenate([source_idx, zpad, target_idx, zpad]) \
        .astype(jnp.int32).reshape(2 * 64, 1)
    trelp = jnp.concatenate([target_rel, zpad]).astype(jnp.int32) \
        .reshape(64, 1)
    gidf = graph_ids.astype(jnp.float32).reshape(N, 1)
    outp = _n6(node_out, inode, g_out, gidf, selp, trelp, rel_table,
               comm_mlp[:D], comm_mlp[D:2 * D], comm_mlp[2 * D:],
               lin1_w, lin1_b.reshape(1, 16), lin2_w, lin2_b.reshape(1, 1))
    return outp[:B]
